# baseline (speedup 1.0000x reference)
import functools

import jax
import jax.numpy as jnp
from jax import lax
from jax.experimental import pallas as pl
from jax.experimental.pallas import tpu as pltpu
from jax.experimental.pallas import tpu_sc as plsc

N = 10000
E = 320000
D = 128
LANES = 16
CHUNK = 128
NC = 2
NS = 16
NW = NC * NS
ITERS = 80
PADCHUNKS = ITERS * NW
E_PAD = PADCHUNKS * CHUNK
RBLK = 80
NRBLK = N // RBLK
RITERS = -(-NRBLK // NS)
NBUF = 2


def _mm_body(x_ref, w_ref, b_ref, o_ref):
    o_ref[...] = (
        jnp.dot(x_ref[...], w_ref[...], preferred_element_type=jnp.float32)
        + b_ref[...]
    )


def _matmul_bias(x, W, b):
    M = x.shape[0]
    BM = 1000
    return pl.pallas_call(
        _mm_body,
        grid=(M // BM,),
        in_specs=[
            pl.BlockSpec((BM, D), lambda i: (i, 0)),
            pl.BlockSpec((D, D), lambda i: (0, 0)),
            pl.BlockSpec((1, D), lambda i: (0, 0)),
        ],
        out_specs=pl.BlockSpec((BM, D), lambda i: (i, 0)),
        out_shape=jax.ShapeDtypeStruct((M, D), jnp.float32),
    )(x, W, b.reshape(1, D))


def _add_body(a_ref, b_ref, o_ref):
    o_ref[...] = a_ref[...] + b_ref[...]


def _add2(a, b):
    BM = 1000
    return pl.pallas_call(
        _add_body,
        grid=(N // BM,),
        in_specs=[pl.BlockSpec((BM, D), lambda i: (i, 0))] * 2,
        out_specs=pl.BlockSpec((BM, D), lambda i: (i, 0)),
        out_shape=jax.ShapeDtypeStruct((N, D), jnp.float32),
    )(a, b)


def _sc_scatter(h, edata):
    mesh = plsc.VectorSubcoreMesh(core_axis_name="c", subcore_axis_name="s")

    @functools.partial(
        pl.kernel,
        out_type=jax.ShapeDtypeStruct((NC, N, D), jnp.float32),
        mesh=mesh,
        compiler_params=pltpu.CompilerParams(needs_layout_passes=False),
        scratch_types=(
            [pltpu.VMEM((3, CHUNK), jnp.int32)] * NBUF
            + [pltpu.VMEM((CHUNK,), jnp.int32)] * NBUF
            + [pltpu.VMEM((CHUNK, D), jnp.float32)] * NBUF
            + [
                pltpu.VMEM((CHUNK,), jnp.int32),
                pltpu.VMEM_SHARED((N, D), jnp.float32),
                pltpu.SemaphoreType.DMA,
                pltpu.SemaphoreType.DMA,
                pltpu.SemaphoreType.DMA,
                pltpu.SemaphoreType.DMA,
            ]
        ),
    )
    def k(h_hbm, edata_hbm, out_hbm,
          ed0, ed1, rv0, rv1, mg0, mg1,
          vv_v, acc, esem0, esem1, gsem0, gsem1):
        edata_v = (ed0, ed1)
        rv_v = (rv0, rv1)
        msgs_v = (mg0, mg1)
        esem = (esem0, esem1)
        gsem = (gsem0, gsem1)
        cid = lax.axis_index("c")
        sid = lax.axis_index("s")
        w = sid * NC + cid

        def chunk_of(i):
            return w + i * NW

        def start_edata(i, mm):
            pltpu.async_copy(
                edata_hbm.at[chunk_of(i)], edata_v[mm], esem[mm])

        def wait_edata(mm):
            pltpu.make_async_copy(
                edata_hbm.at[0], edata_v[mm], esem[mm]).wait()

        def start_gather(mm):
            pltpu.async_copy(
                h_hbm.at[edata_v[mm].at[1]], msgs_v[mm], gsem[mm])

        def wait_gather(mm):
            pltpu.make_async_copy(
                h_hbm.at[edata_v[mm].at[1]], msgs_v[mm], gsem[mm]).wait()

        def sync_scatter(mm):
            pltpu.sync_copy(msgs_v[mm], acc.at[rv_v[mm]], add=True)

        def unpack_edata(mm):
            for j in range(CHUNK // LANES):
                sl = pl.ds(j * LANES, LANES)
                rv_v[mm][sl] = edata_v[mm][0, sl]
                vv_v[sl] = edata_v[mm][2, sl]

        def scale(mm):
            def scale_body(g, carry):
                msgs = msgs_v[mm]
                for u in range(4):
                    e = g * 4 + u
                    v = plsc.bitcast(
                        plsc.load_gather(
                            vv_v, [jnp.full((LANES,), e, jnp.int32)]),
                        jnp.float32)
                    for j in range(D // LANES):
                        fsl = pl.ds(j * LANES, LANES)
                        msgs[e, fsl] = msgs[e, fsl] * v
                return carry

            lax.fori_loop(0, CHUNK // 4, scale_body, 0)

        for m in range(NBUF):
            start_edata(m, m)

        def zero_body(r, carry):
            for j in range(D // LANES):
                mg0[r, pl.ds(j * LANES, LANES)] = jnp.zeros(
                    (LANES,), jnp.float32)
            return carry

        lax.fori_loop(0, RBLK, zero_body, 0)
        for t in range(RITERS):
            rb = sid + t * NS

            @pl.when(rb < NRBLK)
            def _():
                r0 = pl.multiple_of(rb * RBLK, 8)
                pltpu.sync_copy(
                    mg0.at[pl.ds(0, RBLK)], acc.at[pl.ds(r0, RBLK)])

        wait_edata(0)
        start_gather(0)

        plsc.subcore_barrier()

        def slot(i, m):
            m1 = (m + 1) % NBUF
            wait_edata(m1)
            start_gather(m1)
            wait_gather(m)
            unpack_edata(m)
            start_edata(i + NBUF, m)
            scale(m)
            sync_scatter(m)

        def pair_body(i2, carry):
            i0 = i2 * NBUF
            for m in range(NBUF):
                slot(i0 + m, m)
            return carry

        lax.fori_loop(0, ITERS // NBUF - 1, pair_body, 0)

        for m in range(NBUF):
            i = (ITERS - NBUF) + m
            m1 = (m + 1) % NBUF
            if m < NBUF - 1:
                wait_edata(m1)
                start_gather(m1)
            wait_gather(m)
            unpack_edata(m)
            scale(m)
            sync_scatter(m)

        plsc.subcore_barrier()

        for t in range(RITERS):
            rb = sid + t * NS

            @pl.when(rb < NRBLK)
            def _():
                r0 = pl.multiple_of(rb * RBLK, 8)
                pltpu.sync_copy(
                    acc.at[pl.ds(r0, RBLK)],
                    out_hbm.at[cid, pl.ds(r0, RBLK)],
                )

    return k(h, edata)


def kernel(x, adj_indices, adj_values, W, b):
    h = _matmul_bias(x, W, b)
    pad = E_PAD - E
    rows = jnp.pad(adj_indices[0], (0, pad))
    cols = jnp.pad(adj_indices[1], (0, pad))
    vals = jnp.pad(adj_values, (0, pad))
    edata = jnp.stack(
        [rows.reshape(PADCHUNKS, CHUNK),
         cols.reshape(PADCHUNKS, CHUNK),
         lax.bitcast_convert_type(vals, jnp.int32).reshape(PADCHUNKS, CHUNK)],
        axis=1)
    parts = _sc_scatter(h, edata)
    return _add2(parts[0], parts[1])
